# baseline (speedup 1.0000x reference)
import jax
import jax.numpy as jnp
from jax.experimental import pallas as pl
from jax.experimental.pallas import tpu as pltpu
from jax.experimental.pallas import tpu_sc as plsc

_WINDOW = 128


def kernel(x, table):
    batch, seq = x.shape
    vocab, dim = table.shape
    num_indices = batch * seq
    indices = x.reshape(1, num_indices).astype(jnp.int32)

    mesh = plsc.VectorSubcoreMesh(core_axis_name="core",
                                  subcore_axis_name="subcore")

    @pl.kernel(
        out_type=jax.ShapeDtypeStruct((num_indices, dim), table.dtype),
        mesh=mesh,
        compiler_params=pltpu.CompilerParams(use_tc_tiling_on_sc=False),
    )
    def gather_kernel(table_hbm, idx_hbm, out_hbm):
        def body(idx_vmem, out_vmem):
            pltpu.sync_copy(table_hbm.at[idx_vmem.at[0]], out_vmem)

        pltpu.emit_pipeline(
            body,
            grid=(num_indices // _WINDOW,),
            in_specs=[
                pl.BlockSpec((1, _WINDOW), index_map=lambda i: (0, i)),
            ],
            out_specs=[
                pl.BlockSpec((_WINDOW, dim), index_map=lambda i: (i, 0)),
            ],
            core_axis_name=("core", "subcore"),
            dimension_semantics=(pltpu.PARALLEL,),
        )(idx_hbm, out_hbm)

    out = gather_kernel(table, indices)
    return out.reshape(batch, seq, dim)


# speedup vs baseline: 1.0722x; 1.0722x over previous
import jax
import jax.numpy as jnp
from jax.experimental import pallas as pl
from jax.experimental.pallas import tpu as pltpu
from jax.experimental.pallas import tpu_sc as plsc

_WINDOW = 128
_GATHERS_PER_STEP = 4


def kernel(x, table):
    batch, seq = x.shape
    vocab, dim = table.shape
    num_indices = batch * seq
    rows_per_step = _WINDOW * _GATHERS_PER_STEP
    num_steps = num_indices // rows_per_step
    indices = x.reshape(num_indices // _WINDOW, _WINDOW).astype(jnp.int32)

    mesh = plsc.VectorSubcoreMesh(core_axis_name="core",
                                  subcore_axis_name="subcore")

    @pl.kernel(
        out_type=jax.ShapeDtypeStruct((num_indices, dim), table.dtype),
        mesh=mesh,
        scratch_types=[pltpu.SemaphoreType.DMA],
        compiler_params=pltpu.CompilerParams(use_tc_tiling_on_sc=False),
    )
    def gather_kernel(table_hbm, idx_hbm, out_hbm, sem):
        def body(idx_vmem, out_vmem):
            copies = [
                pltpu.async_copy(
                    table_hbm.at[idx_vmem.at[j]],
                    out_vmem.at[pl.ds(j * _WINDOW, _WINDOW)],
                    sem,
                )
                for j in range(_GATHERS_PER_STEP)
            ]
            for c in copies:
                c.wait()

        pltpu.emit_pipeline(
            body,
            grid=(num_steps,),
            in_specs=[
                pl.BlockSpec((_GATHERS_PER_STEP, _WINDOW),
                             index_map=lambda i: (i, 0)),
            ],
            out_specs=[
                pl.BlockSpec((rows_per_step, dim),
                             index_map=lambda i: (i, 0)),
            ],
            core_axis_name=("core", "subcore"),
            dimension_semantics=(pltpu.PARALLEL,),
        )(idx_hbm, out_hbm)

    out = gather_kernel(table, indices)
    return out.reshape(batch, seq, dim)
